# Initial kernel scaffold; baseline (speedup 1.0000x reference)
import jax
import jax.numpy as jnp
from jax.experimental import pallas as pl


def kernel(x):
    raise NotImplementedError("write your pallas kernel here")


# speedup vs baseline: 9.4077x; 9.4077x over previous
import functools

import jax
import jax.numpy as jnp
from jax import lax
from jax.experimental import pallas as pl
from jax.experimental.pallas import tpu as pltpu
from jax.experimental.pallas import tpu_sc as plsc

L = 100
N = L * L
B = 128
NW = 32
NG = B // 16
ROWW = L * B
UNROLL = 3


def _ising_body(x_hbm, out_hbm, x_ring, s_ring, g_ring, res_v, sems):
    cid = lax.axis_index("c")
    sid = lax.axis_index("s")
    wid = sid * 2 + cid

    r0 = (wid * L) // NW
    r1 = ((wid + 1) * L) // NW

    zf = jnp.zeros((16,), jnp.float32)
    onef = jnp.full((16,), 1.0, jnp.float32)
    negf = jnp.full((16,), -1.0, jnp.float32)

    def dma_row(q):
        phys = lax.select(q >= L, q - L, q)
        slot = q % 2
        return pltpu.async_copy(
            x_hbm.at[pl.ds(phys * L, L)], x_ring.at[slot], sems.at[slot]
        )

    def wait_row(q):
        slot = q % 2
        pltpu.make_async_copy(
            x_hbm.at[pl.ds(0, L)], x_ring.at[slot], sems.at[slot]
        ).wait()

    def sg_unit(xslot, n, g):
        x0 = x_ring[xslot, n, 0, pl.ds(g * 16, 16)]
        x1 = x_ring[xslot, n, 1, pl.ds(g * 16, 16)]
        u = x0 + x1
        d = x0 - x1
        sgn = jnp.where(d < 0.0, onef, negf)
        gv = sgn * u + d
        return u, sgn, gv

    def produce_row(q, acc_s, wsel):
        xslot = q % 2
        sgbase = (q % 2) * ROWW
        out = []
        for g in range(NG):
            def body(n, a, g=g):
                u, sgn, gv = sg_unit(xslot, n, g)
                s_ring[pl.ds(sgbase + n * B + g * 16, 16)] = sgn
                g_ring[pl.ds(sgbase + n * B + g * 16, 16)] = gv
                return a + u * wsel
            out.append(
                plsc.parallel_loop(0, L, unroll=2, carry=acc_s[g])(body)
            )
        return tuple(out)

    def fused_row(r, acc_s, acc_e, acc_t, wsel):
        xslot = (r + 1) % 2
        pbase = ((r + 1) % 2) * ROWW
        cbase = (r % 2) * ROWW
        outs = []
        oute = []
        outt = []
        for g in range(NG):
            go = g * 16
            sc0 = s_ring[pl.ds(cbase + go, 16)]
            gc0 = g_ring[pl.ds(cbase + go, 16)]

            def body(n, carry, g=g, go=go):
                a_s, a_e, a_t, s_c, g_c = carry
                u, s_d, g_d = sg_unit(xslot, n, g)
                s_ring[pl.ds(pbase + n * B + go, 16)] = s_d
                g_ring[pl.ds(pbase + n * B + go, 16)] = g_d
                s_r = s_ring[pl.ds(cbase + (n + 1) * B + go, 16)]
                g_r = g_ring[pl.ds(cbase + (n + 1) * B + go, 16)]
                sn = s_r + s_d
                gn = g_r + g_d
                a_e = a_e + s_c * sn
                a_t = a_t + g_c * sn + s_c * gn
                return a_s + u * wsel, a_e, a_t, s_r, g_r

            a_s, a_e, a_t, s_c, g_c = plsc.parallel_loop(
                0, L - 1, unroll=UNROLL,
                carry=(acc_s[g], acc_e[g], acc_t[g], sc0, gc0),
            )(body)

            u, s_d, g_d = sg_unit(xslot, L - 1, g)
            s_ring[pl.ds(pbase + (L - 1) * B + go, 16)] = s_d
            g_ring[pl.ds(pbase + (L - 1) * B + go, 16)] = g_d
            s_r = s_ring[pl.ds(cbase + go, 16)]
            g_r = g_ring[pl.ds(cbase + go, 16)]
            sn = s_r + s_d
            gn = g_r + g_d
            outs.append(a_s + u * wsel)
            oute.append(a_e + s_c * sn)
            outt.append(a_t + g_c * sn + s_c * gn)
        return tuple(outs), tuple(oute), tuple(outt)

    dma_row(r0)
    dma_row(r0 + 1)
    wait_row(r0)
    acc_s = produce_row(r0, (zf,) * NG, onef)

    def row_step(r, carry):
        acc_s = carry[:NG]
        acc_e = carry[NG : 2 * NG]
        acc_t = carry[2 * NG :]

        wait_row(r + 1)

        @pl.when(r + 2 <= r1)
        def _():
            dma_row(r + 2)

        wsel = lax.broadcast(lax.select(r + 1 < r1, 1.0, 0.0), (16,))
        acc_s, acc_e, acc_t = fused_row(r, acc_s, acc_e, acc_t, wsel)
        return acc_s + acc_e + acc_t

    fin = lax.fori_loop(r0, r1, row_step, acc_s + (zf,) * (2 * NG))
    acc_s, acc_e, acc_t = fin[:NG], fin[NG : 2 * NG], fin[2 * NG :]

    for g in range(NG):
        res_v[0, pl.ds(g * 16, 16)] = acc_s[g]
        res_v[1, pl.ds(g * 16, 16)] = acc_e[g]
        res_v[2, pl.ds(g * 16, 16)] = acc_t[g]
    pltpu.sync_copy(res_v, out_hbm.at[wid])


@jax.jit
def kernel(x):
    xt = jnp.transpose(x, (1, 2, 0))
    run = functools.partial(
        pl.kernel,
        mesh=plsc.VectorSubcoreMesh(core_axis_name="c", subcore_axis_name="s"),
        out_type=jax.ShapeDtypeStruct((NW, 3, B), jnp.float32),
        compiler_params=pltpu.CompilerParams(
            needs_layout_passes=False,
            disable_bounds_checks=True,
            disable_semaphore_checks=True,
        ),
        scratch_types=[
            pltpu.VMEM((2, L, 2, B), jnp.float32),
            pltpu.VMEM((2 * ROWW,), jnp.float32),
            pltpu.VMEM((2 * ROWW,), jnp.float32),
            pltpu.VMEM((3, B), jnp.float32),
            pltpu.SemaphoreType.DMA((2,)),
        ],
    )(_ising_body)
    part = run(xt)
    psum = part.sum(axis=0)
    return (psum[2] - psum[1] * psum[0]) * (1.0 / N)
